# Initial kernel scaffold; baseline (speedup 1.0000x reference)
import jax
import jax.numpy as jnp
from jax.experimental import pallas as pl


def kernel(features, coors):
    raise NotImplementedError("write your pallas kernel here")


# speedup vs baseline: 1.3844x; 1.3844x over previous
import functools

import jax
import jax.numpy as jnp
from jax import lax
from jax.experimental import pallas as pl
from jax.experimental.pallas import tpu as pltpu
from jax.experimental.pallas import tpu_sc as plsc

N = 150000
C = 64
S = 8192
NW = 32
NP = 150016
CH = NP // NW
SEGW = S // NW
LCH = 9376
NGRP = LCH // 16
FB = 128
IB = FB + 16


def _mesh():
    return plsc.VectorSubcoreMesh(
        core_axis_name="c", subcore_axis_name="s", num_cores=2, num_subcores=16
    )


_PARAMS = pltpu.CompilerParams(
    needs_layout_passes=False, use_tc_tiling_on_sc=False
)


@functools.partial(
    pl.kernel,
    out_type=jax.ShapeDtypeStruct((NP,), jnp.int32),
    mesh=_mesh(),
    compiler_params=_PARAMS,
    scratch_types=[
        pltpu.VMEM((CH * 4,), jnp.int32),
        pltpu.VMEM((CH,), jnp.int32),
    ],
)
def _linearize(coors_hbm, lin_hbm, coors_v, lin_v):
    w = lax.axis_index("c") * 16 + lax.axis_index("s")
    base = w * CH
    pltpu.sync_copy(coors_hbm.at[pl.ds(base * 4, CH * 4)], coors_v)

    def body(g, carry):
        flat = g * 64 + lax.iota(jnp.int32, 16) * 4
        b = plsc.load_gather(coors_v, [flat])
        z = plsc.load_gather(coors_v, [flat + 1])
        y = plsc.load_gather(coors_v, [flat + 2])
        x = plsc.load_gather(coors_v, [flat + 3])
        lin_v[pl.ds(g * 16, 16)] = (
            (b << 12) | ((z >> 2) << 8) | ((y >> 2) << 4) | (x >> 2)
        )
        return carry

    lax.fori_loop(0, CH // 16, body, 0)
    pltpu.sync_copy(lin_v, lin_hbm.at[pl.ds(base, CH)])


@functools.partial(
    pl.kernel,
    out_type=jax.ShapeDtypeStruct((S, C), jnp.float32),
    mesh=_mesh(),
    compiler_params=_PARAMS,
    scratch_types=[
        pltpu.VMEM((LCH,), jnp.int32),
        pltpu.VMEM((IB,), jnp.int32),
        pltpu.VMEM((IB,), jnp.int32),
        pltpu.VMEM((FB, C), jnp.float32),
        pltpu.VMEM((SEGW, C), jnp.float32),
        pltpu.SemaphoreType.DMA,
    ],
)
def _segmax(feat_hbm, lin_hbm, out_hbm, lin_v, idxbuf, slobuf, rows_v, acc_v,
            gsem):
    w = lax.axis_index("c") * 16 + lax.axis_index("s")
    lo = w * SEGW
    hi = lo + SEGW
    neginf = jnp.full((16,), -jnp.inf, jnp.float32)
    zero16 = jnp.zeros((16,), jnp.int32)

    def init_acc(i, carry):
        for j in range(C // 16):
            acc_v[i, pl.ds(j * 16, 16)] = neginf
        return carry

    lax.fori_loop(0, SEGW, init_acc, 0)
    for k in range(IB // 16):
        idxbuf[pl.ds(k * 16, 16)] = zero16
        slobuf[pl.ds(k * 16, 16)] = zero16

    def flush(n):
        copies = []
        for k in range(FB // 16):
            iv = idxbuf[pl.ds(k * 16, 16)]
            copies.append(
                pltpu.async_copy(
                    feat_hbm.at[iv], rows_v.at[pl.ds(k * 16, 16)], gsem
                )
            )
        for cp in copies:
            cp.wait()

        def acc_body(i, carry):
            s = slobuf[pl.ds(i, 16)][0]
            for j in range(C // 16):
                r = rows_v[i, pl.ds(j * 16, 16)]
                a = acc_v[s, pl.ds(j * 16, 16)]
                acc_v[s, pl.ds(j * 16, 16)] = jnp.maximum(a, r)
            return carry

        lax.fori_loop(0, n, acc_body, 0)

    def scan_chunk(ch, cursor):
        pltpu.sync_copy(lin_hbm.at[pl.ds(ch * LCH, LCH)], lin_v)
        base_pt = ch * LCH

        def grp(g, cur):
            v = lin_v[pl.ds(g * 16, 16)]
            m = (v >= lo) & (v < hi)
            cnt = jnp.max(plsc.all_reduce_population_count(m))

            @pl.when(cnt > 0)
            def _():
                gidx = base_pt + g * 16 + lax.iota(jnp.int32, 16)
                plsc.store_compressed(idxbuf.at[pl.ds(cur, 16)], gidx, mask=m)
                plsc.store_compressed(slobuf.at[pl.ds(cur, 16)], v - lo, mask=m)

            cur2 = cur + cnt

            @pl.when(cur2 >= FB)
            def _():
                flush(FB)
                lv = idxbuf[pl.ds(FB, 16)]
                idxbuf[pl.ds(0, 16)] = lv
                sv = slobuf[pl.ds(FB, 16)]
                slobuf[pl.ds(0, 16)] = sv

            return lax.select(cur2 >= FB, cur2 - FB, cur2)

        return lax.fori_loop(0, NGRP, grp, cursor)

    cursor = lax.fori_loop(0, NP // LCH, scan_chunk, jnp.int32(0))
    flush(cursor)
    pltpu.sync_copy(acc_v, out_hbm.at[pl.ds(lo, SEGW)])


def kernel(features, coors):
    coors_p = jnp.pad(coors, ((0, NP - N), (0, 0)), constant_values=2)
    lin = _linearize(coors_p.reshape(-1))
    return _segmax(features, lin)


# speedup vs baseline: 1.5279x; 1.1036x over previous
import functools

import jax
import jax.numpy as jnp
from jax import lax
from jax.experimental import pallas as pl
from jax.experimental.pallas import tpu as pltpu
from jax.experimental.pallas import tpu_sc as plsc

N = 150000
C = 64
S = 8192
NW = 32
NP = 150016
CH = NP // NW
SEGW = S // NW
LCH = 9376
NGRP = LCH // 16
FB = 128
IB = FB + 16


def _mesh():
    return plsc.VectorSubcoreMesh(
        core_axis_name="c", subcore_axis_name="s", num_cores=2, num_subcores=16
    )


_PARAMS = pltpu.CompilerParams(
    needs_layout_passes=False, use_tc_tiling_on_sc=False
)


@functools.partial(
    pl.kernel,
    out_type=jax.ShapeDtypeStruct((NP,), jnp.int32),
    mesh=_mesh(),
    compiler_params=_PARAMS,
    scratch_types=[
        pltpu.VMEM((CH * 4,), jnp.int32),
        pltpu.VMEM((CH,), jnp.int32),
    ],
)
def _linearize(coors_hbm, lin_hbm, coors_v, lin_v):
    w = lax.axis_index("c") * 16 + lax.axis_index("s")
    base = w * CH
    pltpu.sync_copy(coors_hbm.at[pl.ds(base * 4, CH * 4)], coors_v)

    def body(g, carry):
        flat = g * 64 + lax.iota(jnp.int32, 16) * 4
        b = plsc.load_gather(coors_v, [flat])
        z = plsc.load_gather(coors_v, [flat + 1])
        y = plsc.load_gather(coors_v, [flat + 2])
        x = plsc.load_gather(coors_v, [flat + 3])
        lin_v[pl.ds(g * 16, 16)] = (
            (b << 12) | ((z >> 2) << 8) | ((y >> 2) << 4) | (x >> 2)
        )
        return carry

    lax.fori_loop(0, CH // 16, body, 0)
    pltpu.sync_copy(lin_v, lin_hbm.at[pl.ds(base, CH)])


@functools.partial(
    pl.kernel,
    out_type=jax.ShapeDtypeStruct((S, C), jnp.float32),
    mesh=_mesh(),
    compiler_params=_PARAMS,
    scratch_types=[
        pltpu.VMEM((LCH,), jnp.int32),
        pltpu.VMEM((IB,), jnp.int32),
        pltpu.VMEM((FB, C), jnp.float32),
        pltpu.VMEM((SEGW, C), jnp.float32),
        pltpu.SemaphoreType.DMA,
    ],
)
def _segmax(feat_hbm, lin_hbm, out_hbm, lin_v, idxbuf, rows_v, acc_v, gsem):
    w = lax.axis_index("c") * 16 + lax.axis_index("s")
    lo = w * SEGW
    hi = lo + SEGW
    neginf = jnp.full((16,), -jnp.inf, jnp.float32)
    zero16 = jnp.zeros((16,), jnp.int32)

    def init_acc(i, carry):
        for j in range(C // 16):
            acc_v[i, pl.ds(j * 16, 16)] = neginf
        return carry

    lax.fori_loop(0, SEGW, init_acc, 0)
    for k in range(IB // 16):
        idxbuf[pl.ds(k * 16, 16)] = zero16

    def flush(n):
        copies = []
        for k in range(FB // 16):
            iv = idxbuf[pl.ds(k * 16, 16)] >> 8
            copies.append(
                pltpu.async_copy(
                    feat_hbm.at[iv], rows_v.at[pl.ds(k * 16, 16)], gsem
                )
            )
        for cp in copies:
            cp.wait()

        def acc_body(i, carry):
            s = idxbuf[pl.ds(i, 16)][0] & 255
            for j in range(C // 16):
                r = rows_v[i, pl.ds(j * 16, 16)]
                a = acc_v[s, pl.ds(j * 16, 16)]
                acc_v[s, pl.ds(j * 16, 16)] = jnp.maximum(a, r)
            return carry

        if isinstance(n, int):
            lax.fori_loop(0, n, acc_body, 0, unroll=2)
        else:
            lax.fori_loop(0, n, acc_body, 0)

    def scan_chunk(ch, carry):
        cursor, giota8 = carry
        pltpu.sync_copy(lin_hbm.at[pl.ds(ch * LCH, LCH)], lin_v)

        def grp(g, carry):
            cur, gi8 = carry
            v = lin_v[pl.ds(g * 16, 16)]
            m = (v >= lo) & (v < hi)
            packed = gi8 | (v - lo)
            plsc.store_compressed(idxbuf.at[pl.ds(cur, 16)], packed, mask=m)
            cnt = plsc.all_reduce_population_count(m)[0]
            cur2 = cur + cnt

            @pl.when(cur2 >= FB)
            def _():
                flush(FB)
                lv = idxbuf[pl.ds(FB, 16)]
                idxbuf[pl.ds(0, 16)] = lv

            return (lax.select(cur2 >= FB, cur2 - FB, cur2), gi8 + (16 << 8))

        return lax.fori_loop(0, NGRP, grp, (cursor, giota8))

    giota8 = lax.iota(jnp.int32, 16) << 8
    cursor, _ = lax.fori_loop(0, NP // LCH, scan_chunk, (jnp.int32(0), giota8))
    flush(cursor)
    pltpu.sync_copy(acc_v, out_hbm.at[pl.ds(lo, SEGW)])


def kernel(features, coors):
    coors_p = jnp.pad(coors, ((0, NP - N), (0, 0)), constant_values=2)
    lin = _linearize(coors_p.reshape(-1))
    return _segmax(features, lin)


# speedup vs baseline: 1.7816x; 1.1661x over previous
import functools

import jax
import jax.numpy as jnp
from jax import lax
from jax.experimental import pallas as pl
from jax.experimental.pallas import tpu as pltpu
from jax.experimental.pallas import tpu_sc as plsc

N = 150000
C = 64
S = 8192
NW = 32
NP = 150016
CH = NP // NW
SEGW = S // NW
LCH = 9376
NBLK = 73
NTAIL = 2
FB = 256
IB = 448


def _mesh():
    return plsc.VectorSubcoreMesh(
        core_axis_name="c", subcore_axis_name="s", num_cores=2, num_subcores=16
    )


_PARAMS = pltpu.CompilerParams(
    needs_layout_passes=False, use_tc_tiling_on_sc=False
)


@functools.partial(
    pl.kernel,
    out_type=jax.ShapeDtypeStruct((NP,), jnp.int32),
    mesh=_mesh(),
    compiler_params=_PARAMS,
    scratch_types=[
        pltpu.VMEM((CH * 4,), jnp.int32),
        pltpu.VMEM((CH,), jnp.int32),
    ],
)
def _linearize(coors_hbm, lin_hbm, coors_v, lin_v):
    w = lax.axis_index("c") * 16 + lax.axis_index("s")
    base = w * CH
    pltpu.sync_copy(coors_hbm.at[pl.ds(base * 4, CH * 4)], coors_v)

    def body(g, carry):
        flat = g * 64 + lax.iota(jnp.int32, 16) * 4
        b = plsc.load_gather(coors_v, [flat])
        z = plsc.load_gather(coors_v, [flat + 1])
        y = plsc.load_gather(coors_v, [flat + 2])
        x = plsc.load_gather(coors_v, [flat + 3])
        lin_v[pl.ds(g * 16, 16)] = (
            (b << 12) | ((z >> 2) << 8) | ((y >> 2) << 4) | (x >> 2)
        )
        return carry

    lax.fori_loop(0, CH // 16, body, 0, unroll=4)
    pltpu.sync_copy(lin_v, lin_hbm.at[pl.ds(base, CH)])


@functools.partial(
    pl.kernel,
    out_type=jax.ShapeDtypeStruct((S, C), jnp.float32),
    mesh=_mesh(),
    compiler_params=_PARAMS,
    scratch_types=[
        pltpu.VMEM((LCH,), jnp.int32),
        pltpu.VMEM((LCH,), jnp.int32),
        pltpu.VMEM((IB,), jnp.int32),
        pltpu.VMEM((FB, C), jnp.float32),
        pltpu.VMEM((SEGW, C), jnp.float32),
        pltpu.SemaphoreType.DMA,
        pltpu.SemaphoreType.DMA,
        pltpu.SemaphoreType.DMA,
    ],
)
def _segmax(feat_hbm, lin_hbm, out_hbm, lin_v0, lin_v1, idxbuf, rows_v,
            acc_v, gsem, lsem0, lsem1):
    w = lax.axis_index("c") * 16 + lax.axis_index("s")
    lo = w * SEGW
    hi = lo + SEGW
    neginf = jnp.full((16,), -jnp.inf, jnp.float32)
    zero16 = jnp.zeros((16,), jnp.int32)

    def init_acc(i, carry):
        for j in range(C // 16):
            acc_v[i, pl.ds(j * 16, 16)] = neginf
        return carry

    lax.fori_loop(0, SEGW, init_acc, 0)
    for k in range(IB // 16):
        idxbuf[pl.ds(k * 16, 16)] = zero16

    def flush(n):
        copies = []
        for k in range(FB // 16):
            iv = idxbuf[pl.ds(k * 16, 16)] >> 8
            copies.append(
                pltpu.async_copy(
                    feat_hbm.at[iv], rows_v.at[pl.ds(k * 16, 16)], gsem
                )
            )
        for cp in copies:
            cp.wait()

        def acc_body(i, carry):
            s = idxbuf[pl.ds(i, 16)][0] & 255
            for j in range(C // 16):
                r = rows_v[i, pl.ds(j * 16, 16)]
                a = acc_v[s, pl.ds(j * 16, 16)]
                acc_v[s, pl.ds(j * 16, 16)] = jnp.maximum(a, r)
            return carry

        if isinstance(n, int):
            lax.fori_loop(0, n, acc_body, 0, unroll=4)
        else:
            lax.fori_loop(0, n, acc_body, 0)

    def scan_block(buf, q, cur, gi8, nt, check_flush):
        for t in range(nt):
            v = buf[pl.ds(q * 128 + t * 16, 16)]
            m = (v >= lo) & (v < hi)
            packed = (gi8 + (t << 12)) | (v - lo)
            plsc.store_compressed(idxbuf.at[pl.ds(cur, 16)], packed, mask=m)
            cur = cur + plsc.all_reduce_population_count(m)[0]
        if check_flush:
            @pl.when(cur >= FB)
            def _():
                flush(FB)
                for k in range((IB - FB) // 16):
                    lv = idxbuf[pl.ds(FB + k * 16, 16)]
                    idxbuf[pl.ds(k * 16, 16)] = lv
            cur = lax.select(cur >= FB, cur - FB, cur)
        return cur

    def scan_chunk(buf, carry):
        cursor, gi8 = carry

        def grp8(q, c):
            cur, g8 = c
            cur = scan_block(buf, q, cur, g8, 8, True)
            return (cur, g8 + (8 << 12))

        cursor, gi8 = lax.fori_loop(0, NBLK, grp8, (cursor, gi8))
        cursor = scan_block(buf, NBLK, cursor, gi8, NTAIL, False)
        return (cursor, gi8 + (NTAIL << 12))

    giota8 = lax.iota(jnp.int32, 16) << 8
    pltpu.async_copy(lin_hbm.at[pl.ds(0, LCH)], lin_v0, lsem0)

    def pair(p, carry):
        pltpu.make_async_copy(lin_hbm.at[pl.ds(0, LCH)], lin_v0, lsem0).wait()
        pltpu.async_copy(
            lin_hbm.at[pl.ds((2 * p + 1) * LCH, LCH)], lin_v1, lsem1
        )
        carry = scan_chunk(lin_v0, carry)
        pltpu.make_async_copy(lin_hbm.at[pl.ds(0, LCH)], lin_v1, lsem1).wait()

        @pl.when(p < (NP // LCH) // 2 - 1)
        def _():
            pltpu.async_copy(
                lin_hbm.at[pl.ds((2 * p + 2) * LCH, LCH)], lin_v0, lsem0
            )

        carry = scan_chunk(lin_v1, carry)
        return carry

    cursor, _ = lax.fori_loop(
        0, (NP // LCH) // 2, pair, (jnp.int32(0), giota8)
    )

    @pl.when(cursor >= FB)
    def _():
        flush(FB)
        for k in range((IB - FB) // 16):
            lv = idxbuf[pl.ds(FB + k * 16, 16)]
            idxbuf[pl.ds(k * 16, 16)] = lv

    cursor = lax.select(cursor >= FB, cursor - FB, cursor)
    flush(cursor)
    pltpu.sync_copy(acc_v, out_hbm.at[pl.ds(lo, SEGW)])


def kernel(features, coors):
    coors_p = jnp.pad(coors, ((0, NP - N), (0, 0)), constant_values=2)
    lin = _linearize(coors_p.reshape(-1))
    return _segmax(features, lin)


# speedup vs baseline: 2.5493x; 1.4310x over previous
import functools

import jax
import jax.numpy as jnp
from jax import lax
from jax.experimental import pallas as pl
from jax.experimental.pallas import tpu as pltpu
from jax.experimental.pallas import tpu_sc as plsc

N = 150000
C = 64
S = 8192
NW = 32
NP = 150016
CH = NP // NW
NV = N - (NW - 1) * CH
SEGW = S // NW
LCH = 9376
NBLK = 73
NTAIL = 2
FB = 256
IB = 448
DUMMY = SEGW


def _mesh():
    return plsc.VectorSubcoreMesh(
        core_axis_name="c", subcore_axis_name="s", num_cores=2, num_subcores=16
    )


_PARAMS = pltpu.CompilerParams(
    needs_layout_passes=False, use_tc_tiling_on_sc=False
)


@functools.partial(
    pl.kernel,
    out_type=jax.ShapeDtypeStruct((NP,), jnp.int32),
    mesh=_mesh(),
    compiler_params=_PARAMS,
    scratch_types=[
        pltpu.VMEM((CH * 4,), jnp.int32),
        pltpu.VMEM((CH,), jnp.int32),
    ],
)
def _linearize(coors_hbm, lin_hbm, coors_v, lin_v):
    w = lax.axis_index("c") * 16 + lax.axis_index("s")
    base = w * CH

    @pl.when(w < NW - 1)
    def _():
        pltpu.sync_copy(coors_hbm.at[pl.ds(base * 4, CH * 4)], coors_v)

    @pl.when(w == NW - 1)
    def _():
        pltpu.sync_copy(
            coors_hbm.at[pl.ds((NW - 1) * CH * 4, NV * 4)],
            coors_v.at[pl.ds(0, NV * 4)],
        )

    def body(g, carry):
        flat = g * 64 + lax.iota(jnp.int32, 16) * 4
        b = plsc.load_gather(coors_v, [flat])
        z = plsc.load_gather(coors_v, [flat + 1])
        y = plsc.load_gather(coors_v, [flat + 2])
        x = plsc.load_gather(coors_v, [flat + 3])
        lin = (b << 12) | ((z >> 2) << 8) | ((y >> 2) << 4) | (x >> 2)
        glob = base + g * 16 + lax.iota(jnp.int32, 16)
        lin_v[pl.ds(g * 16, 16)] = jnp.where(glob < N, lin, S)
        return carry

    lax.fori_loop(0, CH // 16, body, 0, unroll=4)
    pltpu.sync_copy(lin_v, lin_hbm.at[pl.ds(base, CH)])


@functools.partial(
    pl.kernel,
    out_type=jax.ShapeDtypeStruct((S, C), jnp.float32),
    mesh=_mesh(),
    compiler_params=_PARAMS,
    scratch_types=[
        pltpu.VMEM((LCH,), jnp.int32),
        pltpu.VMEM((LCH,), jnp.int32),
        pltpu.VMEM((IB,), jnp.int32),
        pltpu.VMEM((FB, C), jnp.float32),
        pltpu.VMEM((SEGW + 1, C), jnp.float32),
        pltpu.SemaphoreType.DMA,
        pltpu.SemaphoreType.DMA,
        pltpu.SemaphoreType.DMA,
    ],
)
def _segmax(feat_hbm, lin_hbm, out_hbm, lin_v0, lin_v1, idxbuf, rows_v,
            acc_v, gsem, lsem0, lsem1):
    w = lax.axis_index("c") * 16 + lax.axis_index("s")
    lo = w * SEGW
    hi = lo + SEGW
    neginf = jnp.full((16,), -jnp.inf, jnp.float32)
    zero16 = jnp.zeros((16,), jnp.int32)

    def init_acc(i, carry):
        for j in range(C // 16):
            acc_v[i, pl.ds(j * 16, 16)] = neginf
        return carry

    lax.fori_loop(0, SEGW + 1, init_acc, 0)
    for k in range(IB // 16):
        idxbuf[pl.ds(k * 16, 16)] = zero16

    def accumulate(n16):
        def blk(bi, carry):
            pv = idxbuf[pl.ds(bi * 16, 16)]
            sv = pv & 511
            for j in range(16):
                s = sv[j]
                for q in range(C // 16):
                    r = rows_v[bi * 16 + j, pl.ds(q * 16, 16)]
                    a = acc_v[s, pl.ds(q * 16, 16)]
                    acc_v[s, pl.ds(q * 16, 16)] = jnp.maximum(a, r)
            return carry

        lax.fori_loop(0, n16, blk, 0)

    def flush(n16):
        copies = []
        for k in range(FB // 16):
            iv = idxbuf[pl.ds(k * 16, 16)] >> 9
            copies.append(
                pltpu.async_copy(
                    feat_hbm.at[iv], rows_v.at[pl.ds(k * 16, 16)], gsem
                )
            )
        for cp in copies:
            cp.wait()
        accumulate(n16)

    def scan_block(buf, q, cur, gi9, nt, check_flush):
        packeds, masks, offs = [], [], [cur]
        for t in range(nt):
            v = buf[pl.ds(q * 128 + t * 16, 16)]
            m = (v >= lo) & (v < hi)
            packeds.append((gi9 + (t << 13)) | (v - lo))
            masks.append(m)
            offs.append(offs[-1] + plsc.all_reduce_population_count(m)[0])
        for t in range(nt):
            plsc.store_compressed(
                idxbuf.at[pl.ds(offs[t], 16)], packeds[t], mask=masks[t]
            )
        cur = offs[nt]
        if check_flush:
            @pl.when(cur >= FB)
            def _():
                flush(FB // 16)
                for k in range((IB - FB) // 16):
                    lv = idxbuf[pl.ds(FB + k * 16, 16)]
                    idxbuf[pl.ds(k * 16, 16)] = lv
            cur = lax.select(cur >= FB, cur - FB, cur)
        return cur

    def scan_chunk(buf, carry):
        cursor, gi8 = carry

        def grp8(q, c):
            cur, g8 = c
            cur = scan_block(buf, q, cur, g8, 8, True)
            return (cur, g8 + (8 << 13))

        cursor, gi8 = lax.fori_loop(0, NBLK, grp8, (cursor, gi8))
        cursor = scan_block(buf, NBLK, cursor, gi8, NTAIL, False)
        return (cursor, gi8 + (NTAIL << 13))

    giota9 = lax.iota(jnp.int32, 16) << 9
    pltpu.async_copy(lin_hbm.at[pl.ds(0, LCH)], lin_v0, lsem0)

    def pair(p, carry):
        pltpu.make_async_copy(lin_hbm.at[pl.ds(0, LCH)], lin_v0, lsem0).wait()
        pltpu.async_copy(
            lin_hbm.at[pl.ds((2 * p + 1) * LCH, LCH)], lin_v1, lsem1
        )
        carry = scan_chunk(lin_v0, carry)
        pltpu.make_async_copy(lin_hbm.at[pl.ds(0, LCH)], lin_v1, lsem1).wait()

        @pl.when(p < (NP // LCH) // 2 - 1)
        def _():
            pltpu.async_copy(
                lin_hbm.at[pl.ds((2 * p + 2) * LCH, LCH)], lin_v0, lsem0
            )

        carry = scan_chunk(lin_v1, carry)
        return carry

    cursor, _ = lax.fori_loop(
        0, (NP // LCH) // 2, pair, (jnp.int32(0), giota9)
    )

    @pl.when(cursor >= FB)
    def _():
        flush(FB // 16)
        for k in range((IB - FB) // 16):
            lv = idxbuf[pl.ds(FB + k * 16, 16)]
            idxbuf[pl.ds(k * 16, 16)] = lv

    cursor = lax.select(cursor >= FB, cursor - FB, cursor)
    idxbuf[pl.ds(cursor, 16)] = zero16 + DUMMY
    flush((cursor + 15) >> 4)
    pltpu.sync_copy(
        acc_v.at[pl.ds(0, SEGW)], out_hbm.at[pl.ds(lo, SEGW)]
    )


def kernel(features, coors):
    lin = _linearize(coors.reshape(-1))
    return _segmax(features, lin)
